# Initial kernel scaffold; baseline (speedup 1.0000x reference)
import jax
import jax.numpy as jnp
from jax.experimental import pallas as pl


def kernel(x, edge_index, W_emb, b_emb, W_gat, att_src, att_dst, b_gat, W_aff, b_aff, W_act, b_act, W_val, b_val):
    raise NotImplementedError("write your pallas kernel here")


# speedup vs baseline: 16.6725x; 16.6725x over previous
import functools

import jax
import jax.numpy as jnp
from jax import lax
from jax.experimental import pallas as pl
from jax.experimental.pallas import tpu as pltpu
from jax.experimental.pallas import tpu_sc as plsc

N = 10000
D = 128
H = 128
A = 32
E = 320000

NC = 2
NS = 16
L = 16
NW = NC * NS
EPT = E // NW
CHUNK = 2000
NCHUNK = EPT // CHUNK
CGROUPS = CHUNK // L
STRIPE = 640
STRIPE_LAST = N - STRIPE * (NS - 1)
ZROWS = 16

NB = 2048
GRID = (N + NB - 1) // NB

f32 = jnp.float32


def _stage1_body(x_ref, we_ref, be_ref, wg_ref, asr_ref, adr_ref,
                 local_ref, h_ref, as_ref, ad_ref):
    cdims = (((1,), (1,)), ((), ()))
    local = jnp.tanh(lax.dot_general(x_ref[...], we_ref[...], cdims,
                                     preferred_element_type=f32) + be_ref[...])
    h = lax.dot_general(local, wg_ref[...], cdims, preferred_element_type=f32)
    local_ref[...] = local
    h_ref[...] = h
    vdims = (((1,), (0,)), ((), ()))
    as_ref[...] = lax.dot_general(h, asr_ref[...], vdims, preferred_element_type=f32)
    ad_ref[...] = lax.dot_general(h, adr_ref[...], vdims, preferred_element_type=f32)


def _stage3_body(local_ref, s_ref, den_ref, bg_ref, waff_ref, ba_ref,
                 wact_ref, bact_ref, wval_ref, bval_ref, a_ref, v_ref):
    cdims = (((1,), (1,)), ((), ()))
    s = s_ref[0] + s_ref[1]
    den = jnp.sum(den_ref[:, 0, :], axis=0)[:, None]
    intra = jnp.tanh(s / (den + 1e-16) + bg_ref[...])
    hid = jnp.tanh(
        lax.dot_general(local_ref[...], waff_ref[:, :H], cdims,
                        preferred_element_type=f32)
        + lax.dot_general(intra, waff_ref[:, H:], cdims,
                          preferred_element_type=f32)
        + ba_ref[...])
    logits = lax.dot_general(hid, wact_ref[...], cdims,
                             preferred_element_type=f32) + bact_ref[...]
    mx = jnp.max(logits, axis=1, keepdims=True)
    lse = jnp.log(jnp.sum(jnp.exp(logits - mx), axis=1, keepdims=True))
    a_ref[...] = logits - mx - lse
    v_ref[...] = jnp.sum(hid * wval_ref[...], axis=1, keepdims=True) + bval_ref[...]


def _edge_body(src_hbm, dst_hbm, as_hbm, ad_hbm, h_hbm,
               s_out, den_out,
               src_v, dst_v, asv, adv, rows_v, den_v, zero_v, s_sh, sem):
    cid = lax.axis_index("c")
    sid = lax.axis_index("s")
    wid = sid * NC + cid

    base = wid * EPT
    pltpu.sync_copy(as_hbm, asv)
    pltpu.sync_copy(ad_hbm, adv)

    def _zden(i, _):
        den_v[pl.ds(i * L, L)] = jnp.zeros((L,), f32)
        return 0
    lax.fori_loop(0, EPT // L, _zden, 0)

    for r in range(ZROWS):
        for c in range(H // L):
            zero_v[r, pl.ds(c * L, L)] = jnp.zeros((L,), f32)
    for j in range(STRIPE // ZROWS):
        @pl.when(jnp.logical_or(sid < NS - 1, j < STRIPE_LAST // ZROWS))
        def _():
            pltpu.sync_copy(zero_v, s_sh.at[pl.ds(sid * STRIPE + j * ZROWS, ZROWS)])
    plsc.subcore_barrier()

    def _chunk(ci, _):
        pltpu.sync_copy(src_hbm.at[pl.ds(base + ci * CHUNK, CHUNK)], src_v)
        pltpu.sync_copy(dst_hbm.at[pl.ds(base + ci * CHUNK, CHUNK)], dst_v)

        def _group(g, _):
            off = g * L
            src16 = src_v[pl.ds(off, L)]
            dst16 = dst_v[pl.ds(off, L)]
            e = plsc.load_gather(asv, [src16]) + plsc.load_gather(adv, [dst16])
            e = jnp.where(e > 0, e, 0.2 * e)
            w = jnp.exp(e)
            plsc.addupdate_scatter(den_v, [dst16], w)
            pltpu.async_copy(h_hbm.at[src16], rows_v, sem).wait()
            dn = lax.GatherDimensionNumbers(offset_dims=(), collapsed_slice_dims=(0,),
                                            start_index_map=(0,))
            for r in range(L):
                wr = lax.gather(w, jnp.full((L, 1), r, jnp.int32), dn, slice_sizes=(1,),
                                mode=lax.GatherScatterMode.PROMISE_IN_BOUNDS)
                for c in range(H // L):
                    rows_v[r, pl.ds(c * L, L)] = rows_v[r, pl.ds(c * L, L)] * wr
            pltpu.sync_copy(rows_v, s_sh.at[dst16], add=True)
            return 0
        lax.fori_loop(0, CGROUPS, _group, 0)
        return 0
    lax.fori_loop(0, NCHUNK, _chunk, 0)

    plsc.subcore_barrier()

    @pl.when(sid < NS - 1)
    def _():
        pltpu.sync_copy(s_sh.at[pl.ds(sid * STRIPE, STRIPE)],
                        s_out.at[cid, pl.ds(sid * STRIPE, STRIPE)])

    @pl.when(sid == NS - 1)
    def _():
        pltpu.sync_copy(s_sh.at[pl.ds((NS - 1) * STRIPE, STRIPE_LAST)],
                        s_out.at[cid, pl.ds((NS - 1) * STRIPE, STRIPE_LAST)])

    pltpu.sync_copy(den_v, den_out.at[wid, 0])


def _edge_call():
    return functools.partial(
        pl.kernel,
        out_type=[jax.ShapeDtypeStruct((NC, N, H), f32),
                  jax.ShapeDtypeStruct((NW, 1, N), f32)],
        mesh=plsc.VectorSubcoreMesh(core_axis_name="c", subcore_axis_name="s",
                                    num_cores=NC, num_subcores=NS),
        scratch_types=[
        pltpu.VMEM((CHUNK,), jnp.int32),
        pltpu.VMEM((CHUNK,), jnp.int32),
        pltpu.VMEM((N,), f32),
        pltpu.VMEM((N,), f32),
        pltpu.VMEM((L, H), f32),
        pltpu.VMEM((N,), f32),
        pltpu.VMEM((ZROWS, H), f32),
            pltpu.VMEM_SHARED((N, H), f32),
            pltpu.SemaphoreType.DMA,
        ],
        compiler_params=pltpu.CompilerParams(needs_layout_passes=False),
    )


def kernel(x, edge_index, W_emb, b_emb, W_gat, att_src, att_dst, b_gat,
           W_aff, b_aff, W_act, b_act, W_val, b_val):
    full = lambda bs: pl.BlockSpec(bs, lambda i: (0,) * len(bs))

    local, h, a_s, a_d = pl.pallas_call(
        _stage1_body,
        grid=(GRID,),
        in_specs=[
            pl.BlockSpec((NB, D), lambda i: (i, 0)),
            full((H, D)), full((1, H)), full((H, H)),
            full((H, 1)), full((H, 1)),
        ],
        out_specs=[
            pl.BlockSpec((NB, H), lambda i: (i, 0)),
            pl.BlockSpec((NB, H), lambda i: (i, 0)),
            pl.BlockSpec((NB, 1), lambda i: (i, 0)),
            pl.BlockSpec((NB, 1), lambda i: (i, 0)),
        ],
        out_shape=[
            jax.ShapeDtypeStruct((N, H), f32),
            jax.ShapeDtypeStruct((N, H), f32),
            jax.ShapeDtypeStruct((N, 1), f32),
            jax.ShapeDtypeStruct((N, 1), f32),
        ],
    )(x, W_emb, b_emb.reshape(1, H), W_gat,
      att_src.reshape(H, 1), att_dst.reshape(H, 1))

    s_part, den_part = _edge_call()(_edge_body)(
        edge_index[0], edge_index[1],
        a_s.reshape(N), a_d.reshape(N), h)

    a, v = pl.pallas_call(
        _stage3_body,
        grid=(GRID,),
        in_specs=[
            pl.BlockSpec((NB, H), lambda i: (i, 0)),
            pl.BlockSpec((NC, NB, H), lambda i: (0, i, 0)),
            pl.BlockSpec((NW, 1, NB), lambda i: (0, 0, i)),
            full((1, H)), full((H, 2 * H)), full((1, H)),
            full((A, H)), full((1, A)), full((1, H)), full((1, 1)),
        ],
        out_specs=[
            pl.BlockSpec((NB, A), lambda i: (i, 0)),
            pl.BlockSpec((NB, 1), lambda i: (i, 0)),
        ],
        out_shape=[
            jax.ShapeDtypeStruct((N, A), f32),
            jax.ShapeDtypeStruct((N, 1), f32),
        ],
    )(local, s_part, den_part, b_gat.reshape(1, H), W_aff,
      b_aff.reshape(1, H), W_act, b_act.reshape(1, A),
      W_val.reshape(1, H), b_val.reshape(1, 1))

    return (a, v)


# speedup vs baseline: 40.1353x; 2.4073x over previous
import functools

import jax
import jax.numpy as jnp
from jax import lax
from jax.experimental import pallas as pl
from jax.experimental.pallas import tpu as pltpu
from jax.experimental.pallas import tpu_sc as plsc

N = 10000
D = 128
H = 128
A = 32
E = 320000

NC = 2
NS = 16
L = 16
NW = NC * NS
EPT = E // NW
CHUNK = 2000
NCHUNK = EPT // CHUNK
CGROUPS = CHUNK // L
NBUF = 5
PF = 3
STRIPE = 640
STRIPE_LAST = N - STRIPE * (NS - 1)
ZROWS = 16

NB = 2048
GRID = (N + NB - 1) // NB

f32 = jnp.float32


def _stage1_body(x_ref, we_ref, be_ref, wg_ref, asr_ref, adr_ref,
                 local_ref, h_ref, as_ref, ad_ref):
    cdims = (((1,), (1,)), ((), ()))
    local = jnp.tanh(lax.dot_general(x_ref[...], we_ref[...], cdims,
                                     preferred_element_type=f32) + be_ref[...])
    h = lax.dot_general(local, wg_ref[...], cdims, preferred_element_type=f32)
    local_ref[...] = local
    h_ref[...] = h
    vdims = (((1,), (0,)), ((), ()))
    as_ref[...] = lax.dot_general(h, asr_ref[...], vdims, preferred_element_type=f32)
    ad_ref[...] = lax.dot_general(h, adr_ref[...], vdims, preferred_element_type=f32)


def _stage3_body(local_ref, s_ref, den_ref, bg_ref, waff_ref, ba_ref,
                 wact_ref, bact_ref, wval_ref, bval_ref, a_ref, v_ref):
    cdims = (((1,), (1,)), ((), ()))
    s = s_ref[0] + s_ref[1]
    den = jnp.sum(den_ref[:, 0, :], axis=0)[:, None]
    intra = jnp.tanh(s / (den + 1e-16) + bg_ref[...])
    hid = jnp.tanh(
        lax.dot_general(local_ref[...], waff_ref[:, :H], cdims,
                        preferred_element_type=f32)
        + lax.dot_general(intra, waff_ref[:, H:], cdims,
                          preferred_element_type=f32)
        + ba_ref[...])
    logits = lax.dot_general(hid, wact_ref[...], cdims,
                             preferred_element_type=f32) + bact_ref[...]
    mx = jnp.max(logits, axis=1, keepdims=True)
    lse = jnp.log(jnp.sum(jnp.exp(logits - mx), axis=1, keepdims=True))
    a_ref[...] = logits - mx - lse
    v_ref[...] = jnp.sum(hid * wval_ref[...], axis=1, keepdims=True) + bval_ref[...]


def _edge_body(src_hbm, dst_hbm, as_hbm, ad_hbm, h_hbm,
               s_out, den_out,
               src_v, dst_v, asv, adv, den_v, zero_v,
               r0, r1, r2, r3, r4, s_sh,
               g0, g1, g2, g3, g4, ss0, ss1, ss2, ss3, ss4):
    rows = (r0, r1, r2, r3, r4)
    gsem = (g0, g1, g2, g3, g4)
    ssem = (ss0, ss1, ss2, ss3, ss4)
    cid = lax.axis_index("c")
    sid = lax.axis_index("s")
    wid = sid * NC + cid
    zidx = jnp.zeros((L,), jnp.int32)
    dn = lax.GatherDimensionNumbers(offset_dims=(), collapsed_slice_dims=(0,),
                                    start_index_map=(0,))

    base = wid * EPT
    pltpu.sync_copy(as_hbm, asv)
    pltpu.sync_copy(ad_hbm, adv)

    def _zden(i, _):
        den_v[pl.ds(i * L, L)] = jnp.zeros((L,), f32)
        return 0
    lax.fori_loop(0, EPT // L, _zden, 0)

    for r in range(ZROWS):
        for c in range(H // L):
            zero_v[r, pl.ds(c * L, L)] = jnp.zeros((L,), f32)

    for j in range(STRIPE // ZROWS):
        @pl.when(jnp.logical_or(sid < NS - 1, j < STRIPE_LAST // ZROWS))
        def _():
            pltpu.sync_copy(zero_v, s_sh.at[pl.ds(sid * STRIPE + j * ZROWS, ZROWS)])
    plsc.subcore_barrier()

    def _chunk(ci, _):
        pltpu.sync_copy(src_hbm.at[pl.ds(base + ci * CHUNK, CHUNK)], src_v)
        pltpu.sync_copy(dst_hbm.at[pl.ds(base + ci * CHUNK, CHUNK)], dst_v)

        for b in range(PF):
            src16 = src_v[pl.ds(b * L, L)]

            @pl.when(ci > 0)
            def _():
                pltpu.make_async_copy(rows[b], s_sh.at[zidx], ssem[b]).wait()
            pltpu.async_copy(h_hbm.at[src16], rows[b], gsem[b])

        def _outer(o, _):
            for b5 in range(NBUF):
                g = o * NBUF + b5
                off = g * L
                src16 = src_v[pl.ds(off, L)]
                dst16 = dst_v[pl.ds(off, L)]
                e = plsc.load_gather(asv, [src16]) + plsc.load_gather(adv, [dst16])
                e = jnp.where(e > 0, e, 0.2 * e)
                w = jnp.exp(e)
                plsc.addupdate_scatter(den_v, [dst16], w)
                pltpu.make_async_copy(h_hbm.at[src16], rows[b5], gsem[b5]).wait()
                for r in range(L):
                    wr = lax.gather(w, jnp.full((L, 1), r, jnp.int32), dn,
                                    slice_sizes=(1,),
                                    mode=lax.GatherScatterMode.PROMISE_IN_BOUNDS)
                    for c in range(H // L):
                        rows[b5][r, pl.ds(c * L, L)] = rows[b5][r, pl.ds(c * L, L)] * wr
                pltpu.async_copy(rows[b5], s_sh.at[dst16], ssem[b5], add=True)

                b2 = (b5 + PF) % NBUF

                @pl.when(g < CGROUPS - PF)
                def _():
                    src16b = src_v[pl.ds(off + PF * L, L)]

                    @pl.when(jnp.logical_or(ci > 0, g >= PF - 1))
                    def _():
                        pltpu.make_async_copy(rows[b2], s_sh.at[zidx], ssem[b2]).wait()
                    pltpu.async_copy(h_hbm.at[src16b], rows[b2], gsem[b2])
            return 0
        lax.fori_loop(0, CGROUPS // NBUF, _outer, 0)
        return 0
    lax.fori_loop(0, NCHUNK, _chunk, 0)

    for b in range(NBUF):
        pltpu.make_async_copy(rows[b], s_sh.at[zidx], ssem[b]).wait()

    plsc.subcore_barrier()

    @pl.when(sid < NS - 1)
    def _():
        pltpu.sync_copy(s_sh.at[pl.ds(sid * STRIPE, STRIPE)],
                        s_out.at[cid, pl.ds(sid * STRIPE, STRIPE)])

    @pl.when(sid == NS - 1)
    def _():
        pltpu.sync_copy(s_sh.at[pl.ds((NS - 1) * STRIPE, STRIPE_LAST)],
                        s_out.at[cid, pl.ds((NS - 1) * STRIPE, STRIPE_LAST)])

    pltpu.sync_copy(den_v, den_out.at[wid, 0])


def _edge_call():
    return functools.partial(
        pl.kernel,
        out_type=[jax.ShapeDtypeStruct((NC, N, H), f32),
                  jax.ShapeDtypeStruct((NW, 1, N), f32)],
        mesh=plsc.VectorSubcoreMesh(core_axis_name="c", subcore_axis_name="s",
                                    num_cores=NC, num_subcores=NS),
        scratch_types=[
            pltpu.VMEM((CHUNK,), jnp.int32),
            pltpu.VMEM((CHUNK,), jnp.int32),
            pltpu.VMEM((N,), f32),
            pltpu.VMEM((N,), f32),
            pltpu.VMEM((N,), f32),
            pltpu.VMEM((ZROWS, H), f32),
            pltpu.VMEM((L, H), f32),
            pltpu.VMEM((L, H), f32),
            pltpu.VMEM((L, H), f32),
            pltpu.VMEM((L, H), f32),
            pltpu.VMEM((L, H), f32),
            pltpu.VMEM_SHARED((N, H), f32),
            pltpu.SemaphoreType.DMA,
            pltpu.SemaphoreType.DMA,
            pltpu.SemaphoreType.DMA,
            pltpu.SemaphoreType.DMA,
            pltpu.SemaphoreType.DMA,
            pltpu.SemaphoreType.DMA,
            pltpu.SemaphoreType.DMA,
            pltpu.SemaphoreType.DMA,
            pltpu.SemaphoreType.DMA,
            pltpu.SemaphoreType.DMA,
        ],
        compiler_params=pltpu.CompilerParams(needs_layout_passes=False),
    )


def kernel(x, edge_index, W_emb, b_emb, W_gat, att_src, att_dst, b_gat,
           W_aff, b_aff, W_act, b_act, W_val, b_val):
    full = lambda bs: pl.BlockSpec(bs, lambda i: (0,) * len(bs))

    local, h, a_s, a_d = pl.pallas_call(
        _stage1_body,
        grid=(GRID,),
        in_specs=[
            pl.BlockSpec((NB, D), lambda i: (i, 0)),
            full((H, D)), full((1, H)), full((H, H)),
            full((H, 1)), full((H, 1)),
        ],
        out_specs=[
            pl.BlockSpec((NB, H), lambda i: (i, 0)),
            pl.BlockSpec((NB, H), lambda i: (i, 0)),
            pl.BlockSpec((NB, 1), lambda i: (i, 0)),
            pl.BlockSpec((NB, 1), lambda i: (i, 0)),
        ],
        out_shape=[
            jax.ShapeDtypeStruct((N, H), f32),
            jax.ShapeDtypeStruct((N, H), f32),
            jax.ShapeDtypeStruct((N, 1), f32),
            jax.ShapeDtypeStruct((N, 1), f32),
        ],
    )(x, W_emb, b_emb.reshape(1, H), W_gat,
      att_src.reshape(H, 1), att_dst.reshape(H, 1))

    s_part, den_part = _edge_call()(_edge_body)(
        edge_index[0], edge_index[1],
        a_s.reshape(N), a_d.reshape(N), h)

    a, v = pl.pallas_call(
        _stage3_body,
        grid=(GRID,),
        in_specs=[
            pl.BlockSpec((NB, H), lambda i: (i, 0)),
            pl.BlockSpec((NC, NB, H), lambda i: (0, i, 0)),
            pl.BlockSpec((NW, 1, NB), lambda i: (0, 0, i)),
            full((1, H)), full((H, 2 * H)), full((1, H)),
            full((A, H)), full((1, A)), full((1, H)), full((1, 1)),
        ],
        out_specs=[
            pl.BlockSpec((NB, A), lambda i: (i, 0)),
            pl.BlockSpec((NB, 1), lambda i: (i, 0)),
        ],
        out_shape=[
            jax.ShapeDtypeStruct((N, A), f32),
            jax.ShapeDtypeStruct((N, 1), f32),
        ],
    )(local, s_part, den_part, b_gat.reshape(1, H), W_aff,
      b_aff.reshape(1, H), W_act, b_act.reshape(1, A),
      W_val.reshape(1, H), b_val.reshape(1, 1))

    return (a, v)
